# initial kernel scaffold (unmeasured)
import jax
import jax.numpy as jnp
from jax import lax
from jax.experimental import pallas as pl
from jax.experimental.pallas import tpu as pltpu

N_DEV = 8


def _ring_allreduce(kv):
    rows, cols = kv.shape
    chunk = rows // N_DEV

    def body(kv_ref, out_ref, recv_buf, rs_send_sems, rs_recv_sems,
             ag_send_sems, ag_recv_sems):
        me = lax.axis_index("i")
        right = lax.rem(me + 1, N_DEV)

        out_ref[...] = kv_ref[...]

        for s in range(N_DEV - 1):
            send_c = lax.rem(me - s + N_DEV, N_DEV)
            recv_c = lax.rem(me - s - 1 + N_DEV, N_DEV)
            rdma = pltpu.make_async_remote_copy(
                src_ref=out_ref.at[pl.ds(send_c * chunk, chunk), :],
                dst_ref=recv_buf.at[s],
                send_sem=rs_send_sems.at[s],
                recv_sem=rs_recv_sems.at[s],
                device_id=(right,),
                device_id_type=pl.DeviceIdType.MESH,
            )
            rdma.start()
            rdma.wait()
            cur = pl.load(out_ref, (pl.ds(recv_c * chunk, chunk), slice(None)))
            pl.store(out_ref, (pl.ds(recv_c * chunk, chunk), slice(None)),
                     cur + recv_buf[s])

        for t in range(N_DEV - 1):
            send_c = lax.rem(me + 1 - t + N_DEV, N_DEV)
            rdma = pltpu.make_async_remote_copy(
                src_ref=out_ref.at[pl.ds(send_c * chunk, chunk), :],
                dst_ref=out_ref.at[pl.ds(send_c * chunk, chunk), :],
                send_sem=ag_send_sems.at[t],
                recv_sem=ag_recv_sems.at[t],
                device_id=(right,),
                device_id_type=pl.DeviceIdType.MESH,
            )
            rdma.start()
            rdma.wait()

    return pl.pallas_call(
        body,
        out_shape=jax.ShapeDtypeStruct((rows, cols), kv.dtype),
        in_specs=[pl.BlockSpec(memory_space=pltpu.VMEM)],
        out_specs=pl.BlockSpec(memory_space=pltpu.VMEM),
        scratch_shapes=[
            pltpu.VMEM((N_DEV - 1, chunk, cols), kv.dtype),
            pltpu.SemaphoreType.DMA((N_DEV - 1,)),
            pltpu.SemaphoreType.DMA((N_DEV - 1,)),
            pltpu.SemaphoreType.DMA((N_DEV - 1,)),
            pltpu.SemaphoreType.DMA((N_DEV - 1,)),
        ],
        compiler_params=pltpu.CompilerParams(collective_id=0),
    )(kv)


def kernel(x, Wdkv, Wuk, Wuv, Wq, Wqr, Wkr, Wo):
    B, S, D = x.shape
    H, Dh, Dr = 16, 128, 32
    x2 = x[0]

    c = x2 @ Wdkv
    Kp = c @ Wuk
    Vp = c @ Wuv

    kv = jnp.concatenate([Kp, Vp], axis=0)
    kv = _ring_allreduce(kv)
    K = kv[:S].reshape(S, H, Dh)
    V = kv[S:].reshape(S, H, Dh)

    Q = (x2 @ Wq).reshape(S, H, Dh)
    Qr = (x2 @ Wqr).reshape(S, H, Dr)
    Kr = x2 @ Wkr

    scale = (Dh + Dr) ** -0.5
    scores = (jnp.einsum("shd,thd->hst", Q, K)
              + jnp.einsum("shr,tr->hst", Qr, Kr)) * scale
    P = jax.nn.softmax(scores, axis=-1)
    O = jnp.einsum("hst,thd->shd", P, V).reshape(S, H * Dh)
    return (O @ Wo)[None]


# baseline (device time: 525978 ns/iter reference)
import jax
import jax.numpy as jnp
from jax import lax
from jax.experimental import pallas as pl
from jax.experimental.pallas import tpu as pltpu

N_DEV = 8


def _ring_allreduce(kv):
    rows, cols = kv.shape
    chunk = rows // N_DEV

    def body(kv_ref, out_ref, recv_buf, rs_send_sems, rs_recv_sems,
             ag_send_sems, ag_recv_sems):
        me = lax.axis_index("i")
        right = lax.rem(me + 1, N_DEV)

        out_ref[...] = kv_ref[...]

        for s in range(N_DEV - 1):
            send_c = lax.rem(me - s + N_DEV, N_DEV)
            recv_c = lax.rem(me - s - 1 + N_DEV, N_DEV)
            rdma = pltpu.make_async_remote_copy(
                src_ref=out_ref.at[pl.ds(send_c * chunk, chunk), :],
                dst_ref=recv_buf.at[s],
                send_sem=rs_send_sems.at[s],
                recv_sem=rs_recv_sems.at[s],
                device_id=(right,),
                device_id_type=pl.DeviceIdType.MESH,
            )
            rdma.start()
            rdma.wait()
            out_ref[pl.ds(recv_c * chunk, chunk), :] = (
                out_ref[pl.ds(recv_c * chunk, chunk), :] + recv_buf[s]
            )

        for t in range(N_DEV - 1):
            send_c = lax.rem(me + 1 - t + N_DEV, N_DEV)
            rdma = pltpu.make_async_remote_copy(
                src_ref=out_ref.at[pl.ds(send_c * chunk, chunk), :],
                dst_ref=out_ref.at[pl.ds(send_c * chunk, chunk), :],
                send_sem=ag_send_sems.at[t],
                recv_sem=ag_recv_sems.at[t],
                device_id=(right,),
                device_id_type=pl.DeviceIdType.MESH,
            )
            rdma.start()
            rdma.wait()

    return pl.pallas_call(
        body,
        out_shape=jax.ShapeDtypeStruct((rows, cols), kv.dtype),
        in_specs=[pl.BlockSpec(memory_space=pltpu.VMEM)],
        out_specs=pl.BlockSpec(memory_space=pltpu.VMEM),
        scratch_shapes=[
            pltpu.VMEM((N_DEV - 1, chunk, cols), kv.dtype),
            pltpu.SemaphoreType.DMA((N_DEV - 1,)),
            pltpu.SemaphoreType.DMA((N_DEV - 1,)),
            pltpu.SemaphoreType.DMA((N_DEV - 1,)),
            pltpu.SemaphoreType.DMA((N_DEV - 1,)),
        ],
    )(kv)


def kernel(x, Wdkv, Wuk, Wuv, Wq, Wqr, Wkr, Wo):
    B, S, D = x.shape
    H, Dh, Dr = 16, 128, 32
    x2 = x[0]

    c = x2 @ Wdkv
    Kp = c @ Wuk
    Vp = c @ Wuv

    kv = jnp.concatenate([Kp, Vp], axis=0)
    kv = _ring_allreduce(kv)
    K = kv[:S].reshape(S, H, Dh)
    V = kv[S:].reshape(S, H, Dh)

    Q = (x2 @ Wq).reshape(S, H, Dh)
    Qr = (x2 @ Wqr).reshape(S, H, Dr)
    Kr = x2 @ Wkr

    scale = (Dh + Dr) ** -0.5
    scores = (jnp.einsum("shd,thd->hst", Q, K)
              + jnp.einsum("shr,tr->hst", Qr, Kr)) * scale
    P = jax.nn.softmax(scores, axis=-1)
    O = jnp.einsum("hst,thd->shd", P, V).reshape(S, H * Dh)
    return (O @ Wo)[None]


# device time: 255498 ns/iter; 2.0586x vs baseline; 2.0586x over previous
import jax
import jax.numpy as jnp
from jax import lax
from jax.experimental import pallas as pl
from jax.experimental.pallas import tpu as pltpu

N = 8
S = 1024
H, Dh, Dr = 16, 128, 32
HPD = H // N
CW = HPD * Dh


def _rs_kv(kv):

    def body(kv_ref, out_ref, rbuf_cw, rbuf_ccw,
             ss_cw, rs_cw, ss_ccw, rs_ccw):
        me = lax.axis_index("i")
        right = lax.rem(me + 1, N)
        left = lax.rem(me - 1 + N, N)

        for s in range(N - 1):
            cs = lax.rem(me - s + N, N)
            cr = lax.rem(me - s - 1 + N, N)
            cw = pltpu.make_async_remote_copy(
                src_ref=kv_ref.at[0, :, pl.ds(cs * CW, CW)],
                dst_ref=rbuf_cw.at[s],
                send_sem=ss_cw.at[s], recv_sem=rs_cw.at[s],
                device_id=(right,), device_id_type=pl.DeviceIdType.MESH,
            )
            ds_ = lax.rem(me + s + 2, N)
            dr = lax.rem(me + s + 3, N)
            ccw = pltpu.make_async_remote_copy(
                src_ref=kv_ref.at[1, :, pl.ds(ds_ * CW, CW)],
                dst_ref=rbuf_ccw.at[s],
                send_sem=ss_ccw.at[s], recv_sem=rs_ccw.at[s],
                device_id=(left,), device_id_type=pl.DeviceIdType.MESH,
            )
            cw.start()
            ccw.start()
            cw.wait()
            ccw.wait()
            kv_ref[0, :, pl.ds(cr * CW, CW)] = (
                kv_ref[0, :, pl.ds(cr * CW, CW)] + rbuf_cw[s])
            kv_ref[1, :, pl.ds(dr * CW, CW)] = (
                kv_ref[1, :, pl.ds(dr * CW, CW)] + rbuf_ccw[s])

        g = lax.rem(me + 1, N)
        out_ref[0] = kv_ref[0, :, pl.ds(g * CW, CW)]
        out_ref[1] = kv_ref[1, :, pl.ds(g * CW, CW)]

    return pl.pallas_call(
        body,
        out_shape=jax.ShapeDtypeStruct((2, S, CW), kv.dtype),
        in_specs=[pl.BlockSpec(memory_space=pltpu.VMEM)],
        out_specs=pl.BlockSpec(memory_space=pltpu.VMEM),
        scratch_shapes=[
            pltpu.VMEM((N - 1, S, CW), kv.dtype),
            pltpu.VMEM((N - 1, S, CW), kv.dtype),
            pltpu.SemaphoreType.DMA((N - 1,)),
            pltpu.SemaphoreType.DMA((N - 1,)),
            pltpu.SemaphoreType.DMA((N - 1,)),
            pltpu.SemaphoreType.DMA((N - 1,)),
        ],
    )(kv)


def _ar_out(op):
    rows, cols = op.shape
    half = rows // 2

    def body(op_ref, out_ref, rbuf_cw, rbuf_ccw,
             rss_cw, rsr_cw, rss_ccw, rsr_ccw,
             ags_cw, agr_cw, ags_ccw, agr_ccw):
        me = lax.axis_index("i")
        right = lax.rem(me + 1, N)
        left = lax.rem(me - 1 + N, N)

        for s in range(N - 1):
            cs = lax.rem(me - s + N, N)
            cr = lax.rem(me - s - 1 + N, N)
            cw = pltpu.make_async_remote_copy(
                src_ref=op_ref.at[pl.ds(0, half), pl.ds(cs * CW, CW)],
                dst_ref=rbuf_cw.at[s],
                send_sem=rss_cw.at[s], recv_sem=rsr_cw.at[s],
                device_id=(right,), device_id_type=pl.DeviceIdType.MESH,
            )
            ds_ = lax.rem(me + s + 2, N)
            dr = lax.rem(me + s + 3, N)
            ccw = pltpu.make_async_remote_copy(
                src_ref=op_ref.at[pl.ds(half, half), pl.ds(ds_ * CW, CW)],
                dst_ref=rbuf_ccw.at[s],
                send_sem=rss_ccw.at[s], recv_sem=rsr_ccw.at[s],
                device_id=(left,), device_id_type=pl.DeviceIdType.MESH,
            )
            cw.start()
            ccw.start()
            cw.wait()
            ccw.wait()
            op_ref[pl.ds(0, half), pl.ds(cr * CW, CW)] = (
                op_ref[pl.ds(0, half), pl.ds(cr * CW, CW)] + rbuf_cw[s])
            op_ref[pl.ds(half, half), pl.ds(dr * CW, CW)] = (
                op_ref[pl.ds(half, half), pl.ds(dr * CW, CW)] + rbuf_ccw[s])

        g = lax.rem(me + 1, N)
        out_ref[:, pl.ds(g * CW, CW)] = op_ref[:, pl.ds(g * CW, CW)]

        for t in range(N - 1):
            cs = lax.rem(me + 1 - t + N, N)
            cw = pltpu.make_async_remote_copy(
                src_ref=out_ref.at[pl.ds(0, half), pl.ds(cs * CW, CW)],
                dst_ref=out_ref.at[pl.ds(0, half), pl.ds(cs * CW, CW)],
                send_sem=ags_cw.at[t], recv_sem=agr_cw.at[t],
                device_id=(right,), device_id_type=pl.DeviceIdType.MESH,
            )
            ds_ = lax.rem(me + t + 1, N)
            ccw = pltpu.make_async_remote_copy(
                src_ref=out_ref.at[pl.ds(half, half), pl.ds(ds_ * CW, CW)],
                dst_ref=out_ref.at[pl.ds(half, half), pl.ds(ds_ * CW, CW)],
                send_sem=ags_ccw.at[t], recv_sem=agr_ccw.at[t],
                device_id=(left,), device_id_type=pl.DeviceIdType.MESH,
            )
            cw.start()
            ccw.start()
            cw.wait()
            ccw.wait()

    return pl.pallas_call(
        body,
        out_shape=jax.ShapeDtypeStruct((rows, cols), op.dtype),
        in_specs=[pl.BlockSpec(memory_space=pltpu.VMEM)],
        out_specs=pl.BlockSpec(memory_space=pltpu.VMEM),
        scratch_shapes=[
            pltpu.VMEM((N - 1, half, CW), op.dtype),
            pltpu.VMEM((N - 1, half, CW), op.dtype),
        ] + [pltpu.SemaphoreType.DMA((N - 1,))] * 8,
    )(op)


def kernel(x, Wdkv, Wuk, Wuv, Wq, Wqr, Wkr, Wo):
    x2 = x[0]

    c = x2 @ Wdkv
    Kp = c @ Wuk
    Vp = c @ Wuv

    kv = jnp.stack([Kp, Vp])
    kv_loc = _rs_kv(kv)
    g = lax.rem(lax.axis_index("i") + 1, N)
    K = kv_loc[0].reshape(S, HPD, Dh)
    V = kv_loc[1].reshape(S, HPD, Dh)

    Wq_loc = lax.dynamic_slice(Wq, (0, g * CW), (Wq.shape[0], CW))
    Wqr_loc = lax.dynamic_slice(Wqr, (0, g * HPD * Dr),
                                (Wqr.shape[0], HPD * Dr))
    Wo_loc = lax.dynamic_slice(Wo, (g * CW, 0), (CW, Wo.shape[1]))

    Q = (x2 @ Wq_loc).reshape(S, HPD, Dh)
    Qr = (x2 @ Wqr_loc).reshape(S, HPD, Dr)
    Kr = x2 @ Wkr

    scale = (Dh + Dr) ** -0.5
    scores = (jnp.einsum("shd,thd->hst", Q, K)
              + jnp.einsum("shr,tr->hst", Qr, Kr)) * scale
    P = jax.nn.softmax(scores, axis=-1)
    O = jnp.einsum("hst,thd->shd", P, V).reshape(S, CW)

    op = O @ Wo_loc
    return _ar_out(op)[None]


# device time: 171666 ns/iter; 3.0640x vs baseline; 1.4883x over previous
import jax
import jax.numpy as jnp
from jax import lax
from jax.experimental import pallas as pl
from jax.experimental.pallas import tpu as pltpu

N = 8
S = 1024
H, Dh, Dr = 16, 128, 32
HPD = H // N
CW = HPD * Dh
DC = 128


def _gather_c_w(c, Wuk, Wuv):

    def body(c_ref, wuk_ref, wuv_ref, cf_ref, wk_ref, wv_ref,
             c_ss, wk_ss, wv_ss, c_rs, wk_rs, wv_rs):
        me = lax.axis_index("i")

        descs = []
        for k in range(1, N):
            j = lax.rem(me + k, N)
            gj = lax.rem(j + 1, N)
            d_c = pltpu.make_async_remote_copy(
                src_ref=c_ref,
                dst_ref=cf_ref.at[:, pl.ds(me * DC, DC)],
                send_sem=c_ss.at[k - 1], recv_sem=c_rs,
                device_id=(j,), device_id_type=pl.DeviceIdType.MESH,
            )
            d_k = pltpu.make_async_remote_copy(
                src_ref=wuk_ref.at[:, pl.ds(gj * CW, CW)],
                dst_ref=wk_ref.at[pl.ds(me * DC, DC), :],
                send_sem=wk_ss.at[k - 1], recv_sem=wk_rs,
                device_id=(j,), device_id_type=pl.DeviceIdType.MESH,
            )
            d_v = pltpu.make_async_remote_copy(
                src_ref=wuv_ref.at[:, pl.ds(gj * CW, CW)],
                dst_ref=wv_ref.at[pl.ds(me * DC, DC), :],
                send_sem=wv_ss.at[k - 1], recv_sem=wv_rs,
                device_id=(j,), device_id_type=pl.DeviceIdType.MESH,
            )
            d_c.start()
            d_k.start()
            d_v.start()
            descs.extend([d_c, d_k, d_v])

        g = lax.rem(me + 1, N)
        cf_ref[:, pl.ds(me * DC, DC)] = c_ref[...]
        wk_ref[pl.ds(me * DC, DC), :] = wuk_ref[:, pl.ds(g * CW, CW)]
        wv_ref[pl.ds(me * DC, DC), :] = wuv_ref[:, pl.ds(g * CW, CW)]

        c_wait = pltpu.make_async_remote_copy(
            src_ref=c_ref, dst_ref=cf_ref.at[:, pl.ds(me * DC, DC)],
            send_sem=c_ss.at[0], recv_sem=c_rs,
            device_id=(me,), device_id_type=pl.DeviceIdType.MESH,
        )
        wk_wait = pltpu.make_async_remote_copy(
            src_ref=wuk_ref.at[:, pl.ds(0, CW)],
            dst_ref=wk_ref.at[pl.ds(me * DC, DC), :],
            send_sem=wk_ss.at[0], recv_sem=wk_rs,
            device_id=(me,), device_id_type=pl.DeviceIdType.MESH,
        )
        wv_wait = pltpu.make_async_remote_copy(
            src_ref=wuv_ref.at[:, pl.ds(0, CW)],
            dst_ref=wv_ref.at[pl.ds(me * DC, DC), :],
            send_sem=wv_ss.at[0], recv_sem=wv_rs,
            device_id=(me,), device_id_type=pl.DeviceIdType.MESH,
        )
        for _ in range(N - 1):
            c_wait.wait_recv()
            wk_wait.wait_recv()
            wv_wait.wait_recv()
        for d in descs:
            d.wait_send()

    return pl.pallas_call(
        body,
        out_shape=(
            jax.ShapeDtypeStruct((S, N * DC), c.dtype),
            jax.ShapeDtypeStruct((N * DC, CW), Wuk.dtype),
            jax.ShapeDtypeStruct((N * DC, CW), Wuv.dtype),
        ),
        in_specs=[pl.BlockSpec(memory_space=pltpu.VMEM)] * 3,
        out_specs=(pl.BlockSpec(memory_space=pltpu.VMEM),) * 3,
        scratch_shapes=[
            pltpu.SemaphoreType.DMA((N - 1,)),
            pltpu.SemaphoreType.DMA((N - 1,)),
            pltpu.SemaphoreType.DMA((N - 1,)),
            pltpu.SemaphoreType.DMA,
            pltpu.SemaphoreType.DMA,
            pltpu.SemaphoreType.DMA,
        ],
    )(c, Wuk, Wuv)


def _ag_o_gemm(o, Wo):

    def body(o_ref, wo_ref, out_ref, obuf, send_sems, recv_sems):
        me = lax.axis_index("i")
        g = lax.rem(me + 1, N)

        descs = []
        for k in range(1, N):
            j = lax.rem(me + k, N)
            d = pltpu.make_async_remote_copy(
                src_ref=o_ref,
                dst_ref=obuf.at[me],
                send_sem=send_sems.at[k - 1],
                recv_sem=recv_sems.at[me],
                device_id=(j,), device_id_type=pl.DeviceIdType.MESH,
            )
            d.start()
            descs.append(d)

        out_ref[...] = jnp.dot(
            o_ref[...], wo_ref[pl.ds(g * CW, CW), :],
            preferred_element_type=jnp.float32,
        )

        for k in range(1, N):
            j = lax.rem(me + k, N)
            gj = lax.rem(j + 1, N)
            w = pltpu.make_async_remote_copy(
                src_ref=o_ref, dst_ref=obuf.at[j],
                send_sem=send_sems.at[k - 1], recv_sem=recv_sems.at[j],
                device_id=(j,), device_id_type=pl.DeviceIdType.MESH,
            )
            w.wait_recv()
            out_ref[...] = out_ref[...] + jnp.dot(
                obuf[j], wo_ref[pl.ds(gj * CW, CW), :],
                preferred_element_type=jnp.float32,
            )

        for d in descs:
            d.wait_send()

    return pl.pallas_call(
        body,
        out_shape=jax.ShapeDtypeStruct((S, Wo.shape[1]), o.dtype),
        in_specs=[pl.BlockSpec(memory_space=pltpu.VMEM)] * 2,
        out_specs=pl.BlockSpec(memory_space=pltpu.VMEM),
        scratch_shapes=[
            pltpu.VMEM((N, S, CW), o.dtype),
            pltpu.SemaphoreType.DMA((N - 1,)),
            pltpu.SemaphoreType.DMA((N,)),
        ],
    )(o, Wo)


def kernel(x, Wdkv, Wuk, Wuv, Wq, Wqr, Wkr, Wo):
    x2 = x[0]

    c = x2 @ Wdkv

    c_full, Wk, Wv = _gather_c_w(c, Wuk, Wuv)

    g = lax.rem(lax.axis_index("i") + 1, N)
    K = (c_full @ Wk).reshape(S, HPD, Dh)
    V = (c_full @ Wv).reshape(S, HPD, Dh)

    Wq_loc = lax.dynamic_slice(Wq, (0, g * CW), (Wq.shape[0], CW))
    Wqr_loc = lax.dynamic_slice(Wqr, (0, g * HPD * Dr),
                                (Wqr.shape[0], HPD * Dr))

    Q = (x2 @ Wq_loc).reshape(S, HPD, Dh)
    Qr = (x2 @ Wqr_loc).reshape(S, HPD, Dr)
    Kr = x2 @ Wkr

    scale = (Dh + Dr) ** -0.5
    scores = (jnp.einsum("shd,thd->hst", Q, K)
              + jnp.einsum("shr,tr->hst", Qr, Kr)) * scale
    P = jax.nn.softmax(scores, axis=-1)
    O = jnp.einsum("hst,thd->shd", P, V).reshape(S, CW)

    return _ag_o_gemm(O, Wo)[None]


# device time: 134121 ns/iter; 3.9217x vs baseline; 1.2799x over previous
import jax
import jax.numpy as jnp
from jax import lax
from jax.experimental import pallas as pl
from jax.experimental.pallas import tpu as pltpu

N = 8
S = 1024
H, Dh, Dr = 16, 128, 32
HPD = H // N
CW = HPD * Dh
DC = 128


def _gather_kv(c, Wuk, Wuv):

    def body(c_ref, wuk_ref, wuv_ref, k_ref, v_ref,
             cbuf, wkbuf, wvbuf, c_ss, wk_ss, wv_ss, c_rs, wk_rs, wv_rs):
        me = lax.axis_index("i")
        g = lax.rem(me + 1, N)

        descs = []
        for k in range(1, N):
            j = lax.rem(me + k, N)
            gj = lax.rem(j + 1, N)
            d_c = pltpu.make_async_remote_copy(
                src_ref=c_ref,
                dst_ref=cbuf.at[me],
                send_sem=c_ss.at[k - 1], recv_sem=c_rs.at[me],
                device_id=(j,), device_id_type=pl.DeviceIdType.MESH,
            )
            d_k = pltpu.make_async_remote_copy(
                src_ref=wuk_ref.at[:, pl.ds(gj * CW, CW)],
                dst_ref=wkbuf.at[me],
                send_sem=wk_ss.at[k - 1], recv_sem=wk_rs.at[me],
                device_id=(j,), device_id_type=pl.DeviceIdType.MESH,
            )
            d_v = pltpu.make_async_remote_copy(
                src_ref=wuv_ref.at[:, pl.ds(gj * CW, CW)],
                dst_ref=wvbuf.at[me],
                send_sem=wv_ss.at[k - 1], recv_sem=wv_rs.at[me],
                device_id=(j,), device_id_type=pl.DeviceIdType.MESH,
            )
            d_c.start()
            d_k.start()
            d_v.start()
            descs.extend([d_c, d_k, d_v])

        k_ref[...] = jnp.dot(c_ref[...], wuk_ref[:, pl.ds(g * CW, CW)],
                             preferred_element_type=jnp.float32)
        v_ref[...] = jnp.dot(c_ref[...], wuv_ref[:, pl.ds(g * CW, CW)],
                             preferred_element_type=jnp.float32)

        for k in range(1, N):
            j = lax.rem(me + k, N)
            wc = pltpu.make_async_remote_copy(
                src_ref=c_ref, dst_ref=cbuf.at[j],
                send_sem=c_ss.at[k - 1], recv_sem=c_rs.at[j],
                device_id=(j,), device_id_type=pl.DeviceIdType.MESH,
            )
            wk_ = pltpu.make_async_remote_copy(
                src_ref=wuk_ref.at[:, pl.ds(0, CW)], dst_ref=wkbuf.at[j],
                send_sem=wk_ss.at[k - 1], recv_sem=wk_rs.at[j],
                device_id=(j,), device_id_type=pl.DeviceIdType.MESH,
            )
            wv_ = pltpu.make_async_remote_copy(
                src_ref=wuv_ref.at[:, pl.ds(0, CW)], dst_ref=wvbuf.at[j],
                send_sem=wv_ss.at[k - 1], recv_sem=wv_rs.at[j],
                device_id=(j,), device_id_type=pl.DeviceIdType.MESH,
            )
            wc.wait_recv()
            wk_.wait_recv()
            k_ref[...] = k_ref[...] + jnp.dot(
                cbuf[j], wkbuf[j], preferred_element_type=jnp.float32)
            wv_.wait_recv()
            v_ref[...] = v_ref[...] + jnp.dot(
                cbuf[j], wvbuf[j], preferred_element_type=jnp.float32)

        for d in descs:
            d.wait_send()

    return pl.pallas_call(
        body,
        out_shape=(
            jax.ShapeDtypeStruct((S, CW), jnp.float32),
            jax.ShapeDtypeStruct((S, CW), jnp.float32),
        ),
        in_specs=[pl.BlockSpec(memory_space=pltpu.VMEM)] * 3,
        out_specs=(pl.BlockSpec(memory_space=pltpu.VMEM),) * 2,
        scratch_shapes=[
            pltpu.VMEM((N, S, DC), c.dtype),
            pltpu.VMEM((N, DC, CW), Wuk.dtype),
            pltpu.VMEM((N, DC, CW), Wuv.dtype),
            pltpu.SemaphoreType.DMA((N - 1,)),
            pltpu.SemaphoreType.DMA((N - 1,)),
            pltpu.SemaphoreType.DMA((N - 1,)),
            pltpu.SemaphoreType.DMA((N,)),
            pltpu.SemaphoreType.DMA((N,)),
            pltpu.SemaphoreType.DMA((N,)),
        ],
    )(c, Wuk, Wuv)


def _ag_o_gemm(o, Wo):

    def body(o_ref, wo_ref, out_ref, obuf, send_sems, recv_sems):
        me = lax.axis_index("i")
        g = lax.rem(me + 1, N)

        descs = []
        for k in range(1, N):
            j = lax.rem(me + k, N)
            d = pltpu.make_async_remote_copy(
                src_ref=o_ref,
                dst_ref=obuf.at[me],
                send_sem=send_sems.at[k - 1],
                recv_sem=recv_sems.at[me],
                device_id=(j,), device_id_type=pl.DeviceIdType.MESH,
            )
            d.start()
            descs.append(d)

        out_ref[...] = jnp.dot(
            o_ref[...], wo_ref[pl.ds(g * CW, CW), :],
            preferred_element_type=jnp.float32,
        )

        for k in range(1, N):
            j = lax.rem(me + k, N)
            gj = lax.rem(j + 1, N)
            w = pltpu.make_async_remote_copy(
                src_ref=o_ref, dst_ref=obuf.at[j],
                send_sem=send_sems.at[k - 1], recv_sem=recv_sems.at[j],
                device_id=(j,), device_id_type=pl.DeviceIdType.MESH,
            )
            w.wait_recv()
            out_ref[...] = out_ref[...] + jnp.dot(
                obuf[j], wo_ref[pl.ds(gj * CW, CW), :],
                preferred_element_type=jnp.float32,
            )

        for d in descs:
            d.wait_send()

    return pl.pallas_call(
        body,
        out_shape=jax.ShapeDtypeStruct((S, Wo.shape[1]), jnp.float32),
        in_specs=[pl.BlockSpec(memory_space=pltpu.VMEM)] * 2,
        out_specs=pl.BlockSpec(memory_space=pltpu.VMEM),
        scratch_shapes=[
            pltpu.VMEM((N, S, CW), o.dtype),
            pltpu.SemaphoreType.DMA((N - 1,)),
            pltpu.SemaphoreType.DMA((N,)),
        ],
    )(o, Wo)


def kernel(x, Wdkv, Wuk, Wuv, Wq, Wqr, Wkr, Wo):
    x2 = x[0]

    c = x2 @ Wdkv

    K_loc, V_loc = _gather_kv(c, Wuk, Wuv)
    K = K_loc.reshape(S, HPD, Dh)
    V = V_loc.reshape(S, HPD, Dh)

    g = lax.rem(lax.axis_index("i") + 1, N)
    Wq_loc = lax.dynamic_slice(Wq, (0, g * CW), (Wq.shape[0], CW))
    Wqr_loc = lax.dynamic_slice(Wqr, (0, g * HPD * Dr),
                                (Wqr.shape[0], HPD * Dr))

    Q = (x2 @ Wq_loc).reshape(S, HPD, Dh)
    Qr = (x2 @ Wqr_loc).reshape(S, HPD, Dr)
    Kr = x2 @ Wkr

    scale = (Dh + Dr) ** -0.5
    scores = (jnp.einsum("shd,thd->hst", Q, K)
              + jnp.einsum("shr,tr->hst", Qr, Kr)) * scale
    P = jax.nn.softmax(scores, axis=-1)
    O = jnp.einsum("hst,thd->shd", P, V).reshape(S, CW)

    return _ag_o_gemm(O.astype(jnp.bfloat16),
                      Wo.astype(jnp.bfloat16))[None]


# device time: 108399 ns/iter; 4.8522x vs baseline; 1.2373x over previous
import jax
import jax.numpy as jnp
from jax import lax
from jax.experimental import pallas as pl
from jax.experimental.pallas import tpu as pltpu

N = 8
S = 1024
H, Dh, Dr = 16, 128, 32
HPD = H // N
CW = HPD * Dh
DC = 128

BF = jnp.bfloat16
F32 = jnp.float32


def _proj_gather(xb, Wdkv, Wuk, Wuv, Wq_loc, Wqr_loc, Wkr):

    def body(x_ref, wdkv_ref, wuk_ref, wuv_ref, wq_ref, wqr_ref, wkr_ref,
             k_ref, v_ref, q_ref, qr_ref, kr_ref,
             cbf, cbuf, wkbuf, wvbuf,
             c_ss, wk_ss, wv_ss, c_rs, wk_rs, wv_rs):
        me = lax.axis_index("i")
        g = lax.rem(me + 1, N)

        cbf[...] = jnp.dot(x_ref[...], wdkv_ref[...],
                           preferred_element_type=F32).astype(BF)

        descs = []
        for k in range(1, N):
            j = lax.rem(me + k, N)
            gj = lax.rem(j + 1, N)
            d_c = pltpu.make_async_remote_copy(
                src_ref=cbf, dst_ref=cbuf.at[me],
                send_sem=c_ss.at[k - 1], recv_sem=c_rs.at[me],
                device_id=(j,), device_id_type=pl.DeviceIdType.MESH,
            )
            d_k = pltpu.make_async_remote_copy(
                src_ref=wuk_ref.at[:, pl.ds(gj * CW, CW)],
                dst_ref=wkbuf.at[me],
                send_sem=wk_ss.at[k - 1], recv_sem=wk_rs.at[me],
                device_id=(j,), device_id_type=pl.DeviceIdType.MESH,
            )
            d_v = pltpu.make_async_remote_copy(
                src_ref=wuv_ref.at[:, pl.ds(gj * CW, CW)],
                dst_ref=wvbuf.at[me],
                send_sem=wv_ss.at[k - 1], recv_sem=wv_rs.at[me],
                device_id=(j,), device_id_type=pl.DeviceIdType.MESH,
            )
            d_c.start()
            d_k.start()
            d_v.start()
            descs.extend([d_c, d_k, d_v])

        q_ref[...] = jnp.dot(x_ref[...], wq_ref[...],
                             preferred_element_type=F32).astype(BF)
        qr_ref[...] = jnp.dot(x_ref[...], wqr_ref[...],
                              preferred_element_type=F32).astype(BF)
        kr_ref[...] = jnp.dot(x_ref[...], wkr_ref[...],
                              preferred_element_type=F32).astype(BF)

        kacc = jnp.dot(cbf[...], wuk_ref[:, pl.ds(g * CW, CW)],
                       preferred_element_type=F32)
        vacc = jnp.dot(cbf[...], wuv_ref[:, pl.ds(g * CW, CW)],
                       preferred_element_type=F32)
        k_ref[...] = kacc.astype(BF)
        v_ref[...] = vacc.astype(BF)

        for k in range(1, N):
            j = lax.rem(me + k, N)
            wc = pltpu.make_async_remote_copy(
                src_ref=cbf, dst_ref=cbuf.at[j],
                send_sem=c_ss.at[k - 1], recv_sem=c_rs.at[j],
                device_id=(j,), device_id_type=pl.DeviceIdType.MESH,
            )
            wk_ = pltpu.make_async_remote_copy(
                src_ref=wuk_ref.at[:, pl.ds(0, CW)], dst_ref=wkbuf.at[j],
                send_sem=wk_ss.at[k - 1], recv_sem=wk_rs.at[j],
                device_id=(j,), device_id_type=pl.DeviceIdType.MESH,
            )
            wv_ = pltpu.make_async_remote_copy(
                src_ref=wuv_ref.at[:, pl.ds(0, CW)], dst_ref=wvbuf.at[j],
                send_sem=wv_ss.at[k - 1], recv_sem=wv_rs.at[j],
                device_id=(j,), device_id_type=pl.DeviceIdType.MESH,
            )
            wc.wait_recv()
            wk_.wait_recv()
            kacc = kacc + jnp.dot(cbuf[j], wkbuf[j],
                                  preferred_element_type=F32)
            k_ref[...] = kacc.astype(BF)
            wv_.wait_recv()
            vacc = vacc + jnp.dot(cbuf[j], wvbuf[j],
                                  preferred_element_type=F32)
            v_ref[...] = vacc.astype(BF)

        for d in descs:
            d.wait_send()

    return pl.pallas_call(
        body,
        out_shape=(
            jax.ShapeDtypeStruct((S, CW), BF),
            jax.ShapeDtypeStruct((S, CW), BF),
            jax.ShapeDtypeStruct((S, CW), BF),
            jax.ShapeDtypeStruct((S, HPD * Dr), BF),
            jax.ShapeDtypeStruct((S, Dr), BF),
        ),
        in_specs=[pl.BlockSpec(memory_space=pltpu.VMEM)] * 7,
        out_specs=(pl.BlockSpec(memory_space=pltpu.VMEM),) * 5,
        scratch_shapes=[
            pltpu.VMEM((S, DC), BF),
            pltpu.VMEM((N, S, DC), BF),
            pltpu.VMEM((N, DC, CW), BF),
            pltpu.VMEM((N, DC, CW), BF),
            pltpu.SemaphoreType.DMA((N - 1,)),
            pltpu.SemaphoreType.DMA((N - 1,)),
            pltpu.SemaphoreType.DMA((N - 1,)),
            pltpu.SemaphoreType.DMA((N,)),
            pltpu.SemaphoreType.DMA((N,)),
            pltpu.SemaphoreType.DMA((N,)),
        ],
    )(xb, Wdkv, Wuk, Wuv, Wq_loc, Wqr_loc, Wkr)


def _ag_o_gemm(o, Wo):

    def body(o_ref, wo_ref, out_ref, obuf, send_sems, recv_sems):
        me = lax.axis_index("i")
        g = lax.rem(me + 1, N)

        descs = []
        for k in range(1, N):
            j = lax.rem(me + k, N)
            d = pltpu.make_async_remote_copy(
                src_ref=o_ref,
                dst_ref=obuf.at[me],
                send_sem=send_sems.at[k - 1],
                recv_sem=recv_sems.at[me],
                device_id=(j,), device_id_type=pl.DeviceIdType.MESH,
            )
            d.start()
            descs.append(d)

        out_ref[...] = jnp.dot(
            o_ref[...], wo_ref[pl.ds(g * CW, CW), :],
            preferred_element_type=F32,
        )

        for k in range(1, N):
            j = lax.rem(me + k, N)
            gj = lax.rem(j + 1, N)
            w = pltpu.make_async_remote_copy(
                src_ref=o_ref, dst_ref=obuf.at[j],
                send_sem=send_sems.at[k - 1], recv_sem=recv_sems.at[j],
                device_id=(j,), device_id_type=pl.DeviceIdType.MESH,
            )
            w.wait_recv()
            out_ref[...] = out_ref[...] + jnp.dot(
                obuf[j], wo_ref[pl.ds(gj * CW, CW), :],
                preferred_element_type=F32,
            )

        for d in descs:
            d.wait_send()

    return pl.pallas_call(
        body,
        out_shape=jax.ShapeDtypeStruct((S, Wo.shape[1]), F32),
        in_specs=[pl.BlockSpec(memory_space=pltpu.VMEM)] * 2,
        out_specs=pl.BlockSpec(memory_space=pltpu.VMEM),
        scratch_shapes=[
            pltpu.VMEM((N, S, CW), BF),
            pltpu.SemaphoreType.DMA((N - 1,)),
            pltpu.SemaphoreType.DMA((N,)),
        ],
    )(o, Wo)


def kernel(x, Wdkv, Wuk, Wuv, Wq, Wqr, Wkr, Wo):
    x2 = x[0]
    g = lax.rem(lax.axis_index("i") + 1, N)

    xb = x2.astype(BF)
    Wq_loc = lax.dynamic_slice(Wq, (0, g * CW), (Wq.shape[0], CW))
    Wqr_loc = lax.dynamic_slice(Wqr, (0, g * HPD * Dr),
                                (Wqr.shape[0], HPD * Dr))

    K_loc, V_loc, Q, Qr, Kr = _proj_gather(
        xb, Wdkv.astype(BF), Wuk.astype(BF), Wuv.astype(BF),
        Wq_loc.astype(BF), Wqr_loc.astype(BF), Wkr.astype(BF))

    K = K_loc.reshape(S, HPD, Dh)
    V = V_loc.reshape(S, HPD, Dh)
    Qh = Q.reshape(S, HPD, Dh)
    Qrh = Qr.reshape(S, HPD, Dr)

    scale = (Dh + Dr) ** -0.5
    scores = (jnp.einsum("shd,thd->hst", Qh, K,
                         preferred_element_type=F32)
              + jnp.einsum("shr,tr->hst", Qrh, Kr,
                           preferred_element_type=F32)) * scale
    P = jax.nn.softmax(scores, axis=-1)
    O = jnp.einsum("hst,thd->shd", P.astype(BF), V,
                   preferred_element_type=F32).reshape(S, CW)

    return _ag_o_gemm(O.astype(BF), Wo.astype(BF))[None]


# device time: 90955 ns/iter; 5.7828x vs baseline; 1.1918x over previous
import jax
import jax.numpy as jnp
from jax import lax
from jax.experimental import pallas as pl
from jax.experimental.pallas import tpu as pltpu

N = 8
S = 1024
H, Dh, Dr = 16, 128, 32
HPD = H // N
CW = HPD * Dh
DC = 128
NB = 4
BS = S // NB

BF = jnp.bfloat16
F32 = jnp.float32


def _proj_gather(xb, Wdkv, Wuk, Wuv, Wq_loc, Wqr0, Wqr1, Wkr):

    def body(x_ref, wdkv_ref, wuk_ref, wuv_ref, wq_ref, wqr0_ref,
             wqr1_ref, wkr_ref,
             k_ref, v_ref, q_ref, qr0_ref, qr1_ref, kr_ref,
             cbf, cbuf, wkbuf, wvbuf,
             c_ss, wk_ss, wv_ss, c_rs, wk_rs, wv_rs):
        me = lax.axis_index("i")
        g = lax.rem(me + 1, N)

        cbf[...] = jnp.dot(x_ref[...], wdkv_ref[...],
                           preferred_element_type=F32).astype(BF)

        descs = []
        for k in range(1, N):
            j = lax.rem(me + k, N)
            gj = lax.rem(j + 1, N)
            d_c = pltpu.make_async_remote_copy(
                src_ref=cbf, dst_ref=cbuf.at[me],
                send_sem=c_ss.at[k - 1], recv_sem=c_rs.at[me],
                device_id=(j,), device_id_type=pl.DeviceIdType.MESH,
            )
            d_k = pltpu.make_async_remote_copy(
                src_ref=wuk_ref.at[:, pl.ds(gj * CW, CW)],
                dst_ref=wkbuf.at[me],
                send_sem=wk_ss.at[k - 1], recv_sem=wk_rs.at[me],
                device_id=(j,), device_id_type=pl.DeviceIdType.MESH,
            )
            d_v = pltpu.make_async_remote_copy(
                src_ref=wuv_ref.at[:, pl.ds(gj * CW, CW)],
                dst_ref=wvbuf.at[me],
                send_sem=wv_ss.at[k - 1], recv_sem=wv_rs.at[me],
                device_id=(j,), device_id_type=pl.DeviceIdType.MESH,
            )
            d_c.start()
            d_k.start()
            d_v.start()
            descs.extend([d_c, d_k, d_v])

        q_ref[...] = jnp.dot(x_ref[...], wq_ref[...],
                             preferred_element_type=F32).astype(BF)
        qr0_ref[...] = jnp.dot(x_ref[...], wqr0_ref[...],
                               preferred_element_type=F32).astype(BF)
        qr1_ref[...] = jnp.dot(x_ref[...], wqr1_ref[...],
                               preferred_element_type=F32).astype(BF)
        kr_ref[...] = jnp.dot(x_ref[...], wkr_ref[...],
                              preferred_element_type=F32).astype(BF)

        kacc = jnp.dot(cbf[...], wuk_ref[:, pl.ds(g * CW, CW)],
                       preferred_element_type=F32)
        vacc = jnp.dot(cbf[...], wuv_ref[:, pl.ds(g * CW, CW)],
                       preferred_element_type=F32)

        for k in range(1, N):
            j = lax.rem(me + k, N)
            wc = pltpu.make_async_remote_copy(
                src_ref=cbf, dst_ref=cbuf.at[j],
                send_sem=c_ss.at[k - 1], recv_sem=c_rs.at[j],
                device_id=(j,), device_id_type=pl.DeviceIdType.MESH,
            )
            wk_ = pltpu.make_async_remote_copy(
                src_ref=wuk_ref.at[:, pl.ds(0, CW)], dst_ref=wkbuf.at[j],
                send_sem=wk_ss.at[k - 1], recv_sem=wk_rs.at[j],
                device_id=(j,), device_id_type=pl.DeviceIdType.MESH,
            )
            wv_ = pltpu.make_async_remote_copy(
                src_ref=wuv_ref.at[:, pl.ds(0, CW)], dst_ref=wvbuf.at[j],
                send_sem=wv_ss.at[k - 1], recv_sem=wv_rs.at[j],
                device_id=(j,), device_id_type=pl.DeviceIdType.MESH,
            )
            wc.wait_recv()
            wk_.wait_recv()
            kacc = kacc + jnp.dot(cbuf[j], wkbuf[j],
                                  preferred_element_type=F32)
            wv_.wait_recv()
            vacc = vacc + jnp.dot(cbuf[j], wvbuf[j],
                                  preferred_element_type=F32)

        k_ref[...] = kacc.astype(BF)
        v_ref[...] = vacc.astype(BF)

        for d in descs:
            d.wait_send()

    return pl.pallas_call(
        body,
        out_shape=(
            jax.ShapeDtypeStruct((S, CW), BF),
            jax.ShapeDtypeStruct((S, CW), BF),
            jax.ShapeDtypeStruct((S, CW), BF),
            jax.ShapeDtypeStruct((S, Dr), BF),
            jax.ShapeDtypeStruct((S, Dr), BF),
            jax.ShapeDtypeStruct((S, Dr), BF),
        ),
        in_specs=[pl.BlockSpec(memory_space=pltpu.VMEM)] * 8,
        out_specs=(pl.BlockSpec(memory_space=pltpu.VMEM),) * 6,
        scratch_shapes=[
            pltpu.VMEM((S, DC), BF),
            pltpu.VMEM((N, S, DC), BF),
            pltpu.VMEM((N, DC, CW), BF),
            pltpu.VMEM((N, DC, CW), BF),
            pltpu.SemaphoreType.DMA((N - 1,)),
            pltpu.SemaphoreType.DMA((N - 1,)),
            pltpu.SemaphoreType.DMA((N - 1,)),
            pltpu.SemaphoreType.DMA((N,)),
            pltpu.SemaphoreType.DMA((N,)),
            pltpu.SemaphoreType.DMA((N,)),
        ],
    )(xb, Wdkv, Wuk, Wuv, Wq_loc, Wqr0, Wqr1, Wkr)


def _attn_ag_gemm(Q, Qr0, Qr1, Kr, K, V, Wo_perm):
    scale = (Dh + Dr) ** -0.5

    def body(q_ref, qr0_ref, qr1_ref, kr_ref, k_ref, v_ref, wo_ref,
             out_ref, obuf, send_sems, recv_sems):
        me = lax.axis_index("i")

        descs = []
        for b in range(NB):
            rows = pl.ds(b * BS, BS)
            for h in range(HPD):
                cols = pl.ds(h * Dh, Dh)
                s = lax.dot_general(
                    q_ref[rows, cols], k_ref[:, cols],
                    (((1,), (1,)), ((), ())),
                    preferred_element_type=F32,
                )
                qr_ref = qr0_ref if h == 0 else qr1_ref
                s = s + lax.dot_general(
                    qr_ref[rows, :], kr_ref[...],
                    (((1,), (1,)), ((), ())),
                    preferred_element_type=F32,
                )
                s = s * scale
                m = jnp.max(s, axis=1, keepdims=True)
                p = jnp.exp(s - m)
                p = (p / jnp.sum(p, axis=1, keepdims=True)).astype(BF)
                ob = jnp.dot(p, v_ref[:, cols],
                             preferred_element_type=F32)
                obuf[rows, pl.ds(me * CW + h * Dh, Dh)] = ob.astype(BF)

            for k in range(1, N):
                j = lax.rem(me + k, N)
                d = pltpu.make_async_remote_copy(
                    src_ref=obuf.at[rows, pl.ds(me * CW, CW)],
                    dst_ref=obuf.at[rows, pl.ds(me * CW, CW)],
                    send_sem=send_sems.at[k - 1, b],
                    recv_sem=recv_sems.at[me, b],
                    device_id=(j,), device_id_type=pl.DeviceIdType.MESH,
                )
                d.start()
                descs.append(d)

        for b in range(NB):
            rows = pl.ds(b * BS, BS)
            for k in range(1, N):
                j = lax.rem(me + k, N)
                w = pltpu.make_async_remote_copy(
                    src_ref=obuf.at[rows, pl.ds(me * CW, CW)],
                    dst_ref=obuf.at[rows, pl.ds(j * CW, CW)],
                    send_sem=send_sems.at[k - 1, b],
                    recv_sem=recv_sems.at[j, b],
                    device_id=(j,), device_id_type=pl.DeviceIdType.MESH,
                )
                w.wait_recv()
            out_ref[rows, :] = jnp.dot(
                obuf[rows, :], wo_ref[...], preferred_element_type=F32)

        for d in descs:
            d.wait_send()

    return pl.pallas_call(
        body,
        out_shape=jax.ShapeDtypeStruct((S, Wo_perm.shape[1]), F32),
        in_specs=[pl.BlockSpec(memory_space=pltpu.VMEM)] * 7,
        out_specs=pl.BlockSpec(memory_space=pltpu.VMEM),
        scratch_shapes=[
            pltpu.VMEM((S, N * CW), BF),
            pltpu.SemaphoreType.DMA((N - 1, NB)),
            pltpu.SemaphoreType.DMA((N, NB)),
        ],
    )(Q, Qr0, Qr1, Kr, K, V, Wo_perm)


def kernel(x, Wdkv, Wuk, Wuv, Wq, Wqr, Wkr, Wo):
    x2 = x[0]
    g = lax.rem(lax.axis_index("i") + 1, N)

    xb = x2.astype(BF)
    Wq_loc = lax.dynamic_slice(Wq, (0, g * CW), (Wq.shape[0], CW))
    Wqr_loc = lax.dynamic_slice(Wqr, (0, g * HPD * Dr),
                                (Wqr.shape[0], HPD * Dr))

    K, V, Q, Qr0, Qr1, Kr = _proj_gather(
        xb, Wdkv.astype(BF), Wuk.astype(BF), Wuv.astype(BF),
        Wq_loc.astype(BF), Wqr_loc[:, :Dr].astype(BF),
        Wqr_loc[:, Dr:].astype(BF), Wkr.astype(BF))

    Wo_perm = jnp.roll(Wo, shift=-CW, axis=0).astype(BF)

    return _attn_ag_gemm(Q, Qr0, Qr1, Kr, K, V, Wo_perm)[None]
